# baseline (speedup 1.0000x reference)
import functools

import jax
import jax.numpy as jnp
from jax import lax
from jax.experimental import pallas as pl
from jax.experimental.pallas import tpu as pltpu
from jax.experimental.pallas import tpu_sc as plsc

N_ROWS = 100000
N_COLS = 100
R = 400
N_CHUNKS = N_ROWS // R
N_WORKERS = 32
N_ITERS = -(-N_CHUNKS // N_WORKERS)
CHUNK_WORDS = R * N_COLS
GROUPS = R // 16

_MESH = plsc.VectorSubcoreMesh(core_axis_name="c", subcore_axis_name="s")


@functools.partial(
    pl.kernel,
    out_type=jax.ShapeDtypeStruct((N_ROWS * N_COLS,), jnp.float32),
    mesh=_MESH,
    compiler_params=pltpu.CompilerParams(needs_layout_passes=False),
    scratch_types=[
        pltpu.VMEM((R,), jnp.int32),
        pltpu.VMEM((R,), jnp.int32),
        pltpu.VMEM((CHUNK_WORDS,), jnp.float32),
        pltpu.VMEM((CHUNK_WORDS,), jnp.float32),
        pltpu.VMEM((R,), jnp.int32),
        pltpu.VMEM((R,), jnp.int32),
        pltpu.SemaphoreType.DMA,
        pltpu.SemaphoreType.DMA,
        pltpu.SemaphoreType.DMA,
        pltpu.SemaphoreType.DMA,
    ],
)
def _sc_onehot(idx_hbm, out_hbm, idx0, idx1, rows0, rows1, offs0, offs1,
               so0, so1, si0, si1):
    wid = lax.axis_index("s") * 2 + lax.axis_index("c")
    bufs = [(idx0, rows0, offs0, so0, si0), (idx1, rows1, offs1, so1, si1)]

    zeros16 = jnp.zeros((16,), jnp.float32)
    ones16 = jnp.ones((16,), jnp.float32)
    lane = lax.iota(jnp.int32, 16)

    def chunk_of(i):
        return jnp.minimum(wid + i * N_WORKERS, N_CHUNKS - 1)

    pending_idx = [
        pltpu.async_copy(idx_hbm.at[pl.ds(chunk_of(i) * R, R)],
                         bufs[i][0], bufs[i][4])
        for i in range(2)
    ]

    def _zero_body(i, _):
        rows0[pl.ds(i * 16, 16)] = zeros16
        rows1[pl.ds(i * 16, 16)] = zeros16
        return 0
    lax.fori_loop(0, CHUNK_WORDS // 16, _zero_body, 0)

    pending_out = [None, None]
    for i in range(N_ITERS):
        b = i % 2
        idx_v, rows_v, offs_v, so, si = bufs[b]
        c = chunk_of(i)
        if pending_out[b] is not None:
            pending_out[b].wait()
            for g in range(GROUPS):
                old = offs_v[pl.ds(g * 16, 16)]
                plsc.store_scatter(rows_v, [old], zeros16)
        pending_idx[b].wait()
        for g in range(GROUPS):
            idx16 = idx_v[pl.ds(g * 16, 16)]
            off = (lane + g * 16) * N_COLS + idx16
            offs_v[pl.ds(g * 16, 16)] = off
            plsc.store_scatter(rows_v, [off], ones16)
        if i + 2 < N_ITERS:
            pending_idx[b] = pltpu.async_copy(
                idx_hbm.at[pl.ds(chunk_of(i + 2) * R, R)], idx_v, si)
        pending_out[b] = pltpu.async_copy(
            rows_v, out_hbm.at[pl.ds(c * CHUNK_WORDS, CHUNK_WORDS)], so)

    for b in range(2):
        if pending_out[b] is not None:
            pending_out[b].wait()


def kernel(atomic_numbers):
    flat = _sc_onehot(atomic_numbers)
    return flat.reshape(N_ROWS, N_COLS)


# speedup vs baseline: 2.2283x; 2.2283x over previous
import functools

import jax
import jax.numpy as jnp
from jax import lax
from jax.experimental import pallas as pl
from jax.experimental.pallas import tpu as pltpu
from jax.experimental.pallas import tpu_sc as plsc

N_ROWS = 100000
N_COLS = 100
R = 400
N_CHUNKS = N_ROWS // R
N_WORKERS = 32
N_ITERS = -(-N_CHUNKS // N_WORKERS)
GROUPS = R // 16

_MESH = plsc.VectorSubcoreMesh(core_axis_name="c", subcore_axis_name="s")


@functools.partial(
    pl.kernel,
    out_type=jax.ShapeDtypeStruct((N_ROWS, N_COLS), jnp.float32),
    mesh=_MESH,
    compiler_params=pltpu.CompilerParams(needs_layout_passes=False),
    scratch_types=[
        pltpu.VMEM((R,), jnp.int32),
        pltpu.VMEM((R,), jnp.int32),
        pltpu.VMEM((R, N_COLS), jnp.float32),
        pltpu.VMEM((R, N_COLS), jnp.float32),
        pltpu.VMEM((R,), jnp.int32),
        pltpu.VMEM((R,), jnp.int32),
        pltpu.SemaphoreType.DMA,
        pltpu.SemaphoreType.DMA,
        pltpu.SemaphoreType.DMA,
        pltpu.SemaphoreType.DMA,
    ],
)
def _sc_onehot(idx_hbm, out_hbm, idx0, idx1, rows0, rows1, offs0, offs1,
               so0, so1, si0, si1):
    wid = lax.axis_index("s") * 2 + lax.axis_index("c")
    bufs = [(idx0, rows0, offs0, so0, si0), (idx1, rows1, offs1, so1, si1)]

    zeros16 = jnp.zeros((16,), jnp.float32)
    ones16 = jnp.ones((16,), jnp.float32)
    lane = lax.iota(jnp.int32, 16)

    def chunk_of(i):
        return jnp.minimum(wid + i * N_WORKERS, N_CHUNKS - 1)

    pending_idx = [
        pltpu.async_copy(idx_hbm.at[pl.ds(chunk_of(i) * R, R)],
                         bufs[i][0], bufs[i][4])
        for i in range(2)
    ]

    def _zero_body(r, _):
        for k in range(6):
            rows0[r, pl.ds(k * 16, 16)] = zeros16
            rows1[r, pl.ds(k * 16, 16)] = zeros16
        rows0[r, pl.ds(N_COLS - 16, 16)] = zeros16
        rows1[r, pl.ds(N_COLS - 16, 16)] = zeros16
        return 0
    lax.fori_loop(0, R, _zero_body, 0)

    pending_out = [None, None]
    for i in range(N_ITERS):
        b = i % 2
        idx_v, rows_v, offs_v, so, si = bufs[b]
        c = chunk_of(i)
        if pending_out[b] is not None:
            pending_out[b].wait()
            for g in range(GROUPS):
                old_col = offs_v[pl.ds(g * 16, 16)]
                plsc.store_scatter(rows_v, [lane + g * 16, old_col], zeros16)
        pending_idx[b].wait()
        for g in range(GROUPS):
            idx16 = idx_v[pl.ds(g * 16, 16)]
            offs_v[pl.ds(g * 16, 16)] = idx16
            plsc.store_scatter(rows_v, [lane + g * 16, idx16], ones16)
        if i + 2 < N_ITERS:
            pending_idx[b] = pltpu.async_copy(
                idx_hbm.at[pl.ds(chunk_of(i + 2) * R, R)], idx_v, si)
        pending_out[b] = pltpu.async_copy(
            rows_v, out_hbm.at[pl.ds(c * R, R)], so)

    for b in range(2):
        if pending_out[b] is not None:
            pending_out[b].wait()


def kernel(atomic_numbers):
    return _sc_onehot(atomic_numbers)
